# Initial kernel scaffold; baseline (speedup 1.0000x reference)
import jax
import jax.numpy as jnp
from jax.experimental import pallas as pl


def kernel(fov, batch_logit_prob, batch_top_k_prob, batch_action_idx, possible_actions, batch_agent_current_pos, step):
    raise NotImplementedError("write your pallas kernel here")


# speedup vs baseline: 3.6534x; 3.6534x over previous
import jax
import jax.numpy as jnp
from jax import lax
from jax.experimental import pallas as pl

H = 64
W = 64
A = 9
ROWS = 256


def _fused_kernel(fov_ref, idx_ref, pos_ref, tab_ref, val_ref,
                  out_ref, pos_out_ref, mask_out_ref):
    fovb = fov_ref[...]
    idx = idx_ref[...]
    pos = pos_ref[...]
    cy = pos[:, 0:1]
    cx = pos[:, 1:2]

    dy = jnp.zeros_like(cy)
    dx = jnp.zeros_like(cx)
    for a in range(A):
        m = idx == a
        dy = jnp.where(m, tab_ref[a, 0], dy)
        dx = jnp.where(m, tab_ref[a, 1], dx)

    ny = jnp.clip(cy + dy, 0, H - 1)
    nx = jnp.clip(cx + dx, 0, W - 1)

    r = fovb.shape[0]
    flat = lax.broadcasted_iota(jnp.int32, (r, H * W), 1)

    f1 = ny * W + nx
    cell = jnp.sum(jnp.where(flat == f1, fovb, 0.0), axis=1, keepdims=True)

    blocked = cell == 1.0
    target = cell == 2.0
    dy2 = jnp.where(blocked, 0, dy)
    dx2 = jnp.where(blocked, 0, dx)
    y2 = cy + dy2
    x2 = cx + dx2
    f2 = jnp.clip(y2, 0, H - 1) * W + jnp.clip(x2, 0, W - 1)

    out_ref[...] = jnp.where(flat == f2, val_ref[0, 0], fovb)
    pos_out_ref[...] = jnp.concatenate([y2, x2], axis=1)
    mask_out_ref[...] = target.astype(jnp.int32)


def _run(fov2d, batch_action_idx, batch_agent_current_pos, possible_actions, val):
    B = fov2d.shape[0]
    grid = (B // ROWS,)
    return pl.pallas_call(
        _fused_kernel,
        grid=grid,
        in_specs=[
            pl.BlockSpec((ROWS, H * W), lambda i: (i, 0)),
            pl.BlockSpec((ROWS, 1), lambda i: (i, 0)),
            pl.BlockSpec((ROWS, 2), lambda i: (i, 0)),
            pl.BlockSpec((A, 2), lambda i: (0, 0)),
            pl.BlockSpec((1, 1), lambda i: (0, 0)),
        ],
        out_specs=[
            pl.BlockSpec((ROWS, H * W), lambda i: (i, 0)),
            pl.BlockSpec((ROWS, 2), lambda i: (i, 0)),
            pl.BlockSpec((ROWS, 1), lambda i: (i, 0)),
        ],
        out_shape=[
            jax.ShapeDtypeStruct((B, H * W), jnp.float32),
            jax.ShapeDtypeStruct((B, 2), jnp.int32),
            jax.ShapeDtypeStruct((B, 1), jnp.int32),
        ],
    )(fov2d, batch_action_idx, batch_agent_current_pos, possible_actions, val)


def kernel(fov, batch_logit_prob, batch_top_k_prob, batch_action_idx,
           possible_actions, batch_agent_current_pos, step):
    B = fov.shape[0]
    val = (3.0 + jnp.asarray(step, jnp.float32)).reshape(1, 1)
    new_fov, new_pos, tmask = _run(
        fov.reshape(B, H * W), batch_action_idx, batch_agent_current_pos,
        possible_actions, val)
    return (new_fov.reshape(B, H, W), new_pos, tmask.reshape(B).astype(bool),
            batch_action_idx, batch_logit_prob, batch_top_k_prob)


# speedup vs baseline: 3.6744x; 1.0057x over previous
import jax
import jax.numpy as jnp
from jax import lax
from jax.experimental import pallas as pl

H = 64
W = 64
A = 9
ROWS = 512


def _fused_kernel(fov_ref, idx_ref, pos_ref, tab_ref, val_ref,
                  out_ref, pos_out_ref, mask_out_ref):
    fovb = fov_ref[...]
    idx = idx_ref[...]
    pos = pos_ref[...]
    cy = pos[:, 0:1]
    cx = pos[:, 1:2]

    dy = jnp.zeros_like(cy)
    dx = jnp.zeros_like(cx)
    for a in range(A):
        m = idx == a
        dy = jnp.where(m, tab_ref[a, 0], dy)
        dx = jnp.where(m, tab_ref[a, 1], dx)

    ny = jnp.clip(cy + dy, 0, H - 1)
    nx = jnp.clip(cx + dx, 0, W - 1)

    r = fovb.shape[0]
    flat = lax.broadcasted_iota(jnp.int32, (r, H * W), 1)

    f1 = ny * W + nx
    cell = jnp.sum(jnp.where(flat == f1, fovb, 0.0), axis=1, keepdims=True)

    blocked = cell == 1.0
    target = cell == 2.0
    dy2 = jnp.where(blocked, 0, dy)
    dx2 = jnp.where(blocked, 0, dx)
    y2 = cy + dy2
    x2 = cx + dx2
    f2 = jnp.clip(y2, 0, H - 1) * W + jnp.clip(x2, 0, W - 1)

    out_ref[...] = jnp.where(flat == f2, val_ref[0, 0], fovb)
    pos_out_ref[...] = jnp.concatenate([y2, x2], axis=1)
    mask_out_ref[...] = target.astype(jnp.int32)


def _run(fov2d, batch_action_idx, batch_agent_current_pos, possible_actions, val):
    B = fov2d.shape[0]
    grid = (B // ROWS,)
    return pl.pallas_call(
        _fused_kernel,
        grid=grid,
        in_specs=[
            pl.BlockSpec((ROWS, H * W), lambda i: (i, 0)),
            pl.BlockSpec((ROWS, 1), lambda i: (i, 0)),
            pl.BlockSpec((ROWS, 2), lambda i: (i, 0)),
            pl.BlockSpec((A, 2), lambda i: (0, 0)),
            pl.BlockSpec((1, 1), lambda i: (0, 0)),
        ],
        out_specs=[
            pl.BlockSpec((ROWS, H * W), lambda i: (i, 0)),
            pl.BlockSpec((ROWS, 2), lambda i: (i, 0)),
            pl.BlockSpec((ROWS, 1), lambda i: (i, 0)),
        ],
        out_shape=[
            jax.ShapeDtypeStruct((B, H * W), jnp.float32),
            jax.ShapeDtypeStruct((B, 2), jnp.int32),
            jax.ShapeDtypeStruct((B, 1), jnp.int32),
        ],
    )(fov2d, batch_action_idx, batch_agent_current_pos, possible_actions, val)


def kernel(fov, batch_logit_prob, batch_top_k_prob, batch_action_idx,
           possible_actions, batch_agent_current_pos, step):
    B = fov.shape[0]
    val = (3.0 + jnp.asarray(step, jnp.float32)).reshape(1, 1)
    new_fov, new_pos, tmask = _run(
        fov.reshape(B, H * W), batch_action_idx, batch_agent_current_pos,
        possible_actions, val)
    return (new_fov.reshape(B, H, W), new_pos, tmask.reshape(B).astype(bool),
            batch_action_idx, batch_logit_prob, batch_top_k_prob)
